# baseline (speedup 1.0000x reference)
import jax
import jax.numpy as jnp
from jax import lax
from jax.experimental import pallas as pl
from jax.experimental.pallas import tpu as pltpu
from jax.experimental.pallas import tpu_sc as plsc

N = 10000
E = 320000
D = 128
R = 4
H = 256
B = 1024

NC = 2
NS = 16
LANES = 16

NODES = 2 * B
SENTINEL = NODES
SLOT_PAD = 10240
ACC_ROWS = 8320
ROWS_PER_SUB = ACC_ROWS // NS
SLOT_PER_SUB = SLOT_PAD // NS
CHUNK = 128
CHUNKS_PER_SUB = 79
EDGES_PER_SUB = CHUNKS_PER_SUB * CHUNK
E_PAD = NC * NS * EDGES_PER_SUB
NODES_PER_SUB = NODES // NS
XN_PER_W = NODES // (NC * NS)
LIST = EDGES_PER_SUB + 2 * CHUNK
TRASH = LIST
LIST_CAP = LIST + LANES
DUMMY_PAIR = SENTINEL * R
PAD_PACK = DUMMY_PAIR << 16
GCH = 64
NBUF = 4


def _sc_body(x_hbm, epk_hbm, dst_hbm, nodes_hbm,
             acc2_hbm, cnt2_hbm, xn_hbm,
             slot_sh, acc_sh, cnt_sh, cl_sh,
             epk_all, dvCA, dvCB, sdA, sdB, pkbuf, posv,
             pkC, svc0, svc1, svc2, svc3, pvc0, pvc1, pvc2, pvc3,
             xr0, xr1, xr2, xr3, padidx, padpk,
             zline, sbuf, ones, nv, vals, g, gp, gph, cv, nv2,
             semA, semB, semG2, semG3, semDA, semDB):
    svcs = (svc0, svc1, svc2, svc3)
    pvcs = (pvc0, pvc1, pvc2, pvc3)
    xrs = (xr0, xr1, xr2, xr3)
    gsems = (semA, semB, semG2, semG3)
    c = lax.axis_index("c")
    s = lax.axis_index("s")
    wid = s * NC + c
    list_base = s * LIST_CAP

    zero16 = jnp.zeros((LANES,), jnp.float32)
    iota16 = lax.iota(jnp.int32, LANES)
    sent16 = jnp.full((LANES,), SENTINEL, jnp.int32)
    one16 = jnp.ones((LANES,), jnp.float32)

    ebase = wid * EDGES_PER_SUB
    pltpu.async_copy(epk_hbm.at[pl.ds(ebase, EDGES_PER_SUB)], epk_all, semDA)

    def zrow_body(i, carry):
        for k in range(D // LANES):
            xr0[i, pl.ds(k * LANES, LANES)] = zero16
        return carry

    lax.fori_loop(0, GCH, zrow_body, 0)
    for k in range(SLOT_PER_SUB // LANES):
        zline[pl.ds(k * LANES, LANES)] = zero16
        sbuf[pl.ds(k * LANES, LANES)] = sent16
    for k in range(CHUNK // LANES):
        ones[pl.ds(k * LANES, LANES)] = one16
        padpk[pl.ds(k * LANES, LANES)] = jnp.full((LANES,), PAD_PACK,
                                                  jnp.int32)

    base_r = s * ROWS_PER_SUB
    for j in range(8):
        pltpu.sync_copy(xr0, acc_sh.at[pl.ds(base_r + j * GCH, GCH)])
    pltpu.sync_copy(xr0.at[pl.ds(0, 8)], acc_sh.at[pl.ds(base_r + 512, 8)])
    pltpu.sync_copy(zline.at[pl.ds(0, ROWS_PER_SUB)],
                    cnt_sh.at[pl.ds(base_r, ROWS_PER_SUB)])
    pltpu.sync_copy(sbuf, slot_sh.at[pl.ds(s * SLOT_PER_SUB, SLOT_PER_SUB)])

    plsc.subcore_barrier()

    nbase = s * NODES_PER_SUB
    pltpu.sync_copy(nodes_hbm.at[pl.ds(nbase, NODES_PER_SUB)], nv)
    for k in range(NODES_PER_SUB // LANES):
        vals[pl.ds(k * LANES, LANES)] = nbase + k * LANES + iota16
    pltpu.sync_copy(vals, slot_sh.at[nv])

    plsc.subcore_barrier()

    pltpu.make_async_copy(epk_hbm.at[pl.ds(0, EDGES_PER_SUB)], epk_all,
                          semDA).wait()

    def load_dv(t, buf, sem):
        pltpu.async_copy(dst_hbm.at[pl.ds(ebase + t * CHUNK, CHUNK)],
                         buf, sem)

    def wait_dv(buf, sem):
        pltpu.make_async_copy(dst_hbm.at[pl.ds(0, CHUNK)], buf, sem).wait()

    def issue_sd(dvC, buf, sem):
        pltpu.async_copy(slot_sh.at[dvC], buf, sem)

    def wait_sd(buf, sem):
        pltpu.make_async_copy(slot_sh.at[dvCA], buf, sem).wait()

    def compact_chunk(t, sdref, cur):
        for k in range(CHUNK // LANES):
            sl = pl.ds(k * LANES, LANES)
            off = t * CHUNK + k * LANES
            sd16 = sdref[sl]
            ep16 = epk_all[pl.ds(off, LANES)]
            ev16 = lax.shift_right_logical(ep16, 16)
            sv16 = ep16 & 0xFFFF
            valid = sd16 < SENTINEL
            pv16 = sd16 * R + ev16
            csum = jnp.where(valid, 1, 0).astype(jnp.int32)
            for sh in (1, 2, 4, 8):
                idx = jnp.maximum(iota16 - sh, 0)
                shifted = jnp.take(csum, idx, mode="wrap")
                csum = csum + jnp.where(iota16 >= sh, shifted, 0)
            pos16 = jnp.where(valid, list_base + cur + csum - 1,
                              list_base + TRASH + iota16)
            pkbuf[sl] = sv16 | lax.shift_left(pv16, 16)
            posv[sl] = pos16
            cur = cur + csum[LANES - 1]
        pltpu.sync_copy(pkbuf, cl_sh.at[posv])
        return cur

    last = CHUNKS_PER_SUB - 1
    load_dv(0, dvCA, semDA)
    wait_dv(dvCA, semDA)
    issue_sd(dvCA, sdA, semA)
    load_dv(1, dvCB, semDB)

    def pairA(i, cur):
        t0 = 2 * i
        wait_dv(dvCB, semDB)
        issue_sd(dvCB, sdB, semB)
        wait_sd(sdA, semA)
        load_dv(jnp.minimum(t0 + 2, last), dvCA, semDA)
        cur = compact_chunk(t0, sdA, cur)
        wait_dv(dvCA, semDA)
        issue_sd(dvCA, sdA, semA)
        wait_sd(sdB, semB)
        load_dv(jnp.minimum(t0 + 3, last), dvCB, semDB)
        cur = compact_chunk(t0 + 1, sdB, cur)
        return cur

    cursor = lax.fori_loop(0, (CHUNKS_PER_SUB - 1) // 2, pairA,
                           jnp.int32(0))
    wait_dv(dvCB, semDB)
    wait_sd(sdA, semA)
    cursor = compact_chunk(last, sdA, cursor)

    for half in range(2):
        for k in range(CHUNK // LANES):
            padidx[pl.ds(k * LANES, LANES)] = (
                list_base + cursor + half * CHUNK + k * LANES + iota16)
        pltpu.sync_copy(padpk, cl_sh.at[padidx])

    ncg = (cursor + GCH - 1) // GCH
    nchunks = jnp.maximum(((ncg + NBUF - 1) // NBUF) * NBUF, NBUF)

    def copy_idx(t, svc, pvc):
        pltpu.sync_copy(cl_sh.at[pl.ds(list_base + t * GCH, GCH)], pkC)
        for k in range(GCH // LANES):
            sl = pl.ds(k * LANES, LANES)
            v = pkC[sl]
            svc[sl] = v & 0xFFFF
            pvc[sl] = lax.shift_right_logical(v, 16)

    def issue_rows(t, j):
        copy_idx(t, svcs[j], pvcs[j])
        pltpu.async_copy(x_hbm.at[svcs[j]], xrs[j], gsems[j])

    def wait_rows(j):
        pltpu.make_async_copy(x_hbm.at[svcs[j]], xrs[j], gsems[j]).wait()

    def scatter_chunk(j):
        pltpu.sync_copy(xrs[j], acc_sh.at[pvcs[j]], add=True)
        pltpu.sync_copy(ones.at[pl.ds(0, GCH)], cnt_sh.at[pvcs[j]], add=True)

    for j in range(NBUF - 1):
        issue_rows(jnp.minimum(j, nchunks - 1), j)

    def quadB(i, carry):
        t0 = NBUF * i
        for j in range(NBUF):
            issue_rows(jnp.minimum(t0 + j + NBUF - 1, nchunks - 1),
                       (j + NBUF - 1) % NBUF)
            wait_rows(j)
            scatter_chunk(j)
        return carry

    lax.fori_loop(0, nchunks // NBUF, quadB, 0)
    for j in range(NBUF - 1):
        wait_rows(j)

    plsc.subcore_barrier()

    pltpu.sync_copy(slot_sh.at[nv], g)
    for r in range(R):
        for k in range(NODES_PER_SUB // LANES):
            sl = pl.ds(k * LANES, LANES)
            gp[sl] = g[sl] * R + r
        fbase = (c * R + r) * NODES + nbase
        for h in range(2):
            for k in range(GCH // LANES):
                sl = pl.ds(k * LANES, LANES)
                gph[sl] = gp[pl.ds(h * GCH + k * LANES, LANES)]
            pltpu.sync_copy(acc_sh.at[gph], xr0)
            pltpu.sync_copy(xr0, acc2_hbm.at[pl.ds(fbase + h * GCH, GCH)])
        pltpu.sync_copy(cnt_sh.at[gp], cv)
        pltpu.sync_copy(cv, cnt2_hbm.at[pl.ds(fbase, NODES_PER_SUB)])

    xb = wid * XN_PER_W
    pltpu.sync_copy(nodes_hbm.at[pl.ds(xb, XN_PER_W)], nv2)
    pltpu.async_copy(x_hbm.at[nv2], xr1, semA).wait()
    pltpu.sync_copy(xr1, xn_hbm.at[pl.ds(xb, XN_PER_W)])


_SC_SCRATCH = [
    pltpu.VMEM_SHARED((SLOT_PAD,), jnp.int32),
    pltpu.VMEM_SHARED((ACC_ROWS, D), jnp.float32),
    pltpu.VMEM_SHARED((ACC_ROWS,), jnp.float32),
    pltpu.VMEM_SHARED((NS * LIST_CAP,), jnp.int32),
    pltpu.VMEM((EDGES_PER_SUB,), jnp.int32),
    pltpu.VMEM((CHUNK,), jnp.int32),
    pltpu.VMEM((CHUNK,), jnp.int32),
    pltpu.VMEM((CHUNK,), jnp.int32),
    pltpu.VMEM((CHUNK,), jnp.int32),
    pltpu.VMEM((CHUNK,), jnp.int32),
    pltpu.VMEM((CHUNK,), jnp.int32),
    pltpu.VMEM((GCH,), jnp.int32),
    pltpu.VMEM((GCH,), jnp.int32),
    pltpu.VMEM((GCH,), jnp.int32),
    pltpu.VMEM((GCH,), jnp.int32),
    pltpu.VMEM((GCH,), jnp.int32),
    pltpu.VMEM((GCH,), jnp.int32),
    pltpu.VMEM((GCH,), jnp.int32),
    pltpu.VMEM((GCH,), jnp.int32),
    pltpu.VMEM((GCH,), jnp.int32),
    pltpu.VMEM((GCH, D), jnp.float32),
    pltpu.VMEM((GCH, D), jnp.float32),
    pltpu.VMEM((GCH, D), jnp.float32),
    pltpu.VMEM((GCH, D), jnp.float32),
    pltpu.VMEM((CHUNK,), jnp.int32),
    pltpu.VMEM((CHUNK,), jnp.int32),
    pltpu.VMEM((SLOT_PER_SUB,), jnp.float32),
    pltpu.VMEM((SLOT_PER_SUB,), jnp.int32),
    pltpu.VMEM((CHUNK,), jnp.float32),
    pltpu.VMEM((NODES_PER_SUB,), jnp.int32),
    pltpu.VMEM((NODES_PER_SUB,), jnp.int32),
    pltpu.VMEM((NODES_PER_SUB,), jnp.int32),
    pltpu.VMEM((NODES_PER_SUB,), jnp.int32),
    pltpu.VMEM((GCH,), jnp.int32),
    pltpu.VMEM((NODES_PER_SUB,), jnp.float32),
    pltpu.VMEM((XN_PER_W,), jnp.int32),
    pltpu.SemaphoreType.DMA,
    pltpu.SemaphoreType.DMA,
    pltpu.SemaphoreType.DMA,
    pltpu.SemaphoreType.DMA,
    pltpu.SemaphoreType.DMA,
    pltpu.SemaphoreType.DMA,
]

_SC_OUT = (
    jax.ShapeDtypeStruct((NC * R * NODES, D), jnp.float32),
    jax.ShapeDtypeStruct((NC * R * NODES,), jnp.float32),
    jax.ShapeDtypeStruct((NODES, D), jnp.float32),
)

_sc_call_cached = None


def _sc_call(*args):
    global _sc_call_cached
    if _sc_call_cached is None:
        _sc_call_cached = pl.kernel(
            _sc_body,
            out_type=_SC_OUT,
            mesh=plsc.VectorSubcoreMesh(core_axis_name="c",
                                        subcore_axis_name="s",
                                        num_cores=NC, num_subcores=NS),
            scratch_types=_SC_SCRATCH,
        )
    return _sc_call_cached(*args)


def _tc_body(acc2_ref, cnt2_ref, xn_ref, wrel_ref, wroot_ref, brg_ref,
             wfc_ref, bfc_ref, out_ref):
    agg = jnp.zeros((NODES, D), jnp.float32)
    for r in range(R):
        acc_r = acc2_ref[0, r] + acc2_ref[1, r]
        cnt_r = cnt2_ref[0, r] + cnt2_ref[1, r]
        norm = 1.0 / jnp.maximum(cnt_r, 1.0)
        agg = agg + jnp.dot(acc_r * norm, wrel_ref[r],
                            preferred_element_type=jnp.float32)
    h = agg + jnp.dot(xn_ref[...], wroot_ref[...],
                      preferred_element_type=jnp.float32) + brg_ref[...]
    h = jnp.maximum(h, 0.0)
    comb = jnp.concatenate([h[:B], h[B:]], axis=1)
    out = jnp.dot(comb, wfc_ref[...],
                  preferred_element_type=jnp.float32) + bfc_ref[...]
    out_ref[...] = jnp.maximum(out, 0.0)


def _tc_call(acc2, cnt2, xn, W_rel, W_root, brg, W_fc, bfc):
    return pl.pallas_call(
        _tc_body,
        out_shape=jax.ShapeDtypeStruct((B, H), jnp.float32),
    )(acc2, cnt2, xn, W_rel, W_root, brg, W_fc, bfc)


def kernel(x, edge_index, edge_type, nest_tensor, food_tensor,
           W_rel, W_root, b_rgcn, W_fc, b_fc):
    src = edge_index[0].astype(jnp.int32)
    dst = edge_index[1].astype(jnp.int32)
    et = edge_type.astype(jnp.int32)
    nodes = jnp.concatenate([nest_tensor, food_tensor]).astype(jnp.int32)
    pad = E_PAD - E
    epk = src | (et << 16)
    epk = jnp.concatenate([epk, jnp.zeros((pad,), jnp.int32)])
    dst = jnp.concatenate([dst, jnp.full((pad,), N, jnp.int32)])

    acc2, cnt2, xn = _sc_call(x, epk, dst, nodes)
    acc2 = acc2.reshape(NC, R, NODES, D)
    cnt2 = cnt2.reshape(NC, R, NODES, 1)
    return _tc_call(acc2, cnt2, xn, W_rel, W_root,
                    b_rgcn.reshape(1, D), W_fc, b_fc.reshape(1, H))


# speedup vs baseline: 2.9214x; 2.9214x over previous
import jax
import jax.numpy as jnp
from jax import lax
from jax.experimental import pallas as pl
from jax.experimental.pallas import tpu as pltpu
from jax.experimental.pallas import tpu_sc as plsc

N = 10000
E = 320000
D = 128
R = 4
H = 256
B = 1024

NC = 2
NS = 16
LANES = 16

NODES = 2 * B
SENTINEL = NODES
SLOT_PAD = 10240
X_ROWS = 10240
XSTAGE = X_ROWS // NS
ACC_ROWS = 2176
ROWS_PER_SUB = ACC_ROWS // NS
SLOT_PER_SUB = SLOT_PAD // NS
CHUNK = 128
CHUNKS_PER_SUB = 79
EDGES_PER_SUB = CHUNKS_PER_SUB * CHUNK
E_PAD = NC * NS * EDGES_PER_SUB
NODES_PER_SUB = NODES // NS
XN_PER_W = NODES // (NC * NS)
GCH = 48
ALLOC_Q = 2 * GCH
LIST = EDGES_PER_SUB + R * ALLOC_Q
TRASH = LIST
LIST_CAP = LIST + LANES
PAD_PACK = SENTINEL << 16


def _sc_body(x_hbm, epk_hbm, dst_hbm, nodes_hbm,
             acc2_hbm, cnt2_hbm, xn_hbm,
             slot_sh, x_sp, acc_sh, cnt_sh, cl_sh,
             dvA, dvB, epA, epB, sdA, sdB, pkbuf, posv,
             padbuf, padpos, pkC, svc0, svc1, pvc0, pvc1,
             xr0, xr1, zline, sbuf, ones, nv, vals, g, gph, cv, nv2,
             semA, semB, semDA, semDB):
    c = lax.axis_index("c")
    s = lax.axis_index("s")
    wid = s * NC + c
    list_base = s * LIST_CAP

    zero16 = jnp.zeros((LANES,), jnp.float32)
    iota16 = lax.iota(jnp.int32, LANES)
    sent16 = jnp.full((LANES,), SENTINEL, jnp.int32)
    one16 = jnp.ones((LANES,), jnp.float32)

    pltpu.async_copy(x_hbm.at[pl.ds(s * XSTAGE, XSTAGE)],
                     x_sp.at[pl.ds(s * XSTAGE, XSTAGE)], semA)

    for k in range(SLOT_PER_SUB // LANES):
        zline[pl.ds(k * LANES, LANES)] = zero16
        sbuf[pl.ds(k * LANES, LANES)] = sent16
    for k in range(GCH // LANES):
        ones[pl.ds(k * LANES, LANES)] = one16
    for k in range(ALLOC_Q // LANES):
        padbuf[pl.ds(k * LANES, LANES)] = jnp.full((LANES,), PAD_PACK,
                                                   jnp.int32)

    pltpu.sync_copy(sbuf, slot_sh.at[pl.ds(s * SLOT_PER_SUB, SLOT_PER_SUB)])
    pltpu.make_async_copy(x_hbm.at[pl.ds(0, XSTAGE)],
                          x_sp.at[pl.ds(0, XSTAGE)], semA).wait()

    plsc.subcore_barrier()

    nbase = s * NODES_PER_SUB
    pltpu.sync_copy(nodes_hbm.at[pl.ds(nbase, NODES_PER_SUB)], nv)
    for k in range(NODES_PER_SUB // LANES):
        vals[pl.ds(k * LANES, LANES)] = nbase + k * LANES + iota16
    pltpu.sync_copy(vals, slot_sh.at[nv])

    plsc.subcore_barrier()

    ebase = wid * EDGES_PER_SUB

    def load_ch(t, dvb, epb, sem):
        pltpu.async_copy(dst_hbm.at[pl.ds(ebase + t * CHUNK, CHUNK)],
                         dvb, sem)
        pltpu.async_copy(epk_hbm.at[pl.ds(ebase + t * CHUNK, CHUNK)],
                         epb, sem)

    def wait_ch(dvb, epb, sem):
        pltpu.make_async_copy(dst_hbm.at[pl.ds(0, CHUNK)], dvb, sem).wait()
        pltpu.make_async_copy(epk_hbm.at[pl.ds(0, CHUNK)], epb, sem).wait()

    def issue_sd(dvb, buf, sem):
        pltpu.async_copy(slot_sh.at[dvb], buf, sem)

    def wait_sd(buf, sem):
        pltpu.make_async_copy(slot_sh.at[dvA], buf, sem).wait()

    last = CHUNKS_PER_SUB - 1

    def prefix16(v):
        out = v
        for sh in (1, 2, 4, 8):
            idx = jnp.maximum(iota16 - sh, 0)
            shifted = jnp.take(out, idx, mode="wrap")
            out = out + jnp.where(iota16 >= sh, shifted, 0)
        return out

    def scan_pass(chunk_fn, carry0):
        load_ch(0, dvA, epA, semDA)
        wait_ch(dvA, epA, semDA)
        issue_sd(dvA, sdA, semA)
        load_ch(1, dvB, epB, semDB)

        def pair(i, carry):
            t0 = 2 * i
            wait_ch(dvB, epB, semDB)
            issue_sd(dvB, sdB, semB)
            wait_sd(sdA, semA)
            load_ch(jnp.minimum(t0 + 2, last), dvA, epA, semDA)
            carry = chunk_fn(sdA, epA, carry)
            wait_ch(dvA, epA, semDA)
            issue_sd(dvA, sdA, semA)
            wait_sd(sdB, semB)
            load_ch(jnp.minimum(t0 + 3, last), dvB, epB, semDB)
            carry = chunk_fn(sdB, epB, carry)
            return carry

        carry = lax.fori_loop(0, (CHUNKS_PER_SUB - 1) // 2, pair, carry0)
        wait_ch(dvB, epB, semDB)
        wait_sd(sdA, semA)
        return chunk_fn(sdA, epA, carry)

    def count_chunk(sdref, epref, carry):
        accA, accB = carry
        for k in range(CHUNK // LANES):
            sl = pl.ds(k * LANES, LANES)
            sd16 = sdref[sl]
            ev16 = lax.shift_right_logical(epref[sl], 16)
            valid = sd16 < SENTINEL
            fld = lax.shift_left(1, (ev16 & 1) * 16)
            accA = accA + jnp.where(valid & (ev16 < 2), fld, 0)
            accB = accB + jnp.where(valid & (ev16 >= 2), fld, 0)
        return (accA, accB)

    zi = jnp.zeros((LANES,), jnp.int32)
    accA, accB = scan_pass(count_chunk, (zi, zi))
    totA = prefix16(accA)[LANES - 1]
    totB = prefix16(accB)[LANES - 1]
    cnts = (totA & 0xFFFF, lax.shift_right_logical(totA, 16),
            totB & 0xFFFF, lax.shift_right_logical(totB, 16))
    allocs = []
    bases = []
    b = jnp.int32(0)
    for r in range(R):
        bases.append(b)
        a = ((jnp.maximum(cnts[r], 1) + ALLOC_Q - 1) // ALLOC_Q) * ALLOC_Q
        allocs.append(a)
        b = b + a

    def compact_chunk(sdref, epref, carry):
        cur0, cur1, cur2, cur3 = carry
        for k in range(CHUNK // LANES):
            sl = pl.ds(k * LANES, LANES)
            sd16 = sdref[sl]
            ep16 = epref[sl]
            ev16 = lax.shift_right_logical(ep16, 16)
            sv16 = ep16 & 0xFFFF
            valid = sd16 < SENTINEL
            fld = lax.shift_left(1, (ev16 & 1) * 16)
            indA = jnp.where(valid & (ev16 < 2), fld, 0)
            indB = jnp.where(valid & (ev16 >= 2), fld, 0)
            csA = prefix16(indA)
            csB = prefix16(indB)
            csel = jnp.where(ev16 < 2, csA, csB)
            cs = lax.shift_right_logical(csel, (ev16 & 1) * 16) & 0xFFFF
            base_l = jnp.where(ev16 < 2,
                               jnp.where(ev16 == 0, cur0, cur1),
                               jnp.where(ev16 == 2, cur2, cur3))
            pos16 = jnp.where(valid, list_base + base_l + cs - 1,
                              list_base + TRASH + iota16)
            pkbuf[sl] = sv16 | lax.shift_left(sd16, 16)
            posv[sl] = pos16
            tA = csA[LANES - 1]
            tB = csB[LANES - 1]
            cur0 = cur0 + (tA & 0xFFFF)
            cur1 = cur1 + lax.shift_right_logical(tA, 16)
            cur2 = cur2 + (tB & 0xFFFF)
            cur3 = cur3 + lax.shift_right_logical(tB, 16)
        pltpu.sync_copy(pkbuf, cl_sh.at[posv])
        return (cur0, cur1, cur2, cur3)

    curs = scan_pass(compact_chunk,
                     (bases[0], bases[1], bases[2], bases[3]))

    for r in range(R):
        lim = bases[r] + allocs[r]
        for k in range(ALLOC_Q // LANES):
            raw = curs[r] + k * LANES + iota16
            padpos[pl.ds(k * LANES, LANES)] = jnp.where(
                raw < lim, list_base + raw, list_base + TRASH + iota16)
        pltpu.sync_copy(padbuf, cl_sh.at[padpos])

    def copy_idx(off, svc, pvc):
        pltpu.sync_copy(cl_sh.at[pl.ds(list_base + off, GCH)], pkC)
        for k in range(GCH // LANES):
            sl = pl.ds(k * LANES, LANES)
            v = pkC[sl]
            svc[sl] = v & 0xFFFF
            pvc[sl] = lax.shift_right_logical(v, 16)

    def issue_rows(svc, buf, sem):
        pltpu.async_copy(x_sp.at[svc], buf, sem)

    def wait_rows(svc, buf, sem):
        pltpu.make_async_copy(x_sp.at[svc], buf, sem).wait()

    def scatter_chunk(xbuf, pvc):
        pltpu.sync_copy(xbuf, acc_sh.at[pvc], add=True)
        pltpu.sync_copy(ones, cnt_sh.at[pvc], add=True)

    def zero_xr0(i, carry):
        for k in range(D // LANES):
            xr0[i, pl.ds(k * LANES, LANES)] = zero16
        return carry

    base_r = s * ROWS_PER_SUB
    pltpu.sync_copy(slot_sh.at[nv], g)

    for r in range(R):
        lax.fori_loop(0, GCH, zero_xr0, 0)
        pltpu.sync_copy(xr0, acc_sh.at[pl.ds(base_r, GCH)])
        pltpu.sync_copy(xr0, acc_sh.at[pl.ds(base_r + GCH, GCH)])
        pltpu.sync_copy(xr0.at[pl.ds(0, ROWS_PER_SUB - 2 * GCH)],
                        acc_sh.at[pl.ds(base_r + 2 * GCH,
                                        ROWS_PER_SUB - 2 * GCH)])
        pltpu.sync_copy(zline.at[pl.ds(0, ROWS_PER_SUB)],
                        cnt_sh.at[pl.ds(base_r, ROWS_PER_SUB)])
        plsc.subcore_barrier()

        nch = allocs[r] // GCH
        b0 = bases[r]
        copy_idx(b0, svc0, pvc0)
        issue_rows(svc0, xr0, semA)

        def pairB(i, carry):
            t0 = 2 * i
            copy_idx(b0 + (t0 + 1) * GCH, svc1, pvc1)
            issue_rows(svc1, xr1, semB)
            wait_rows(svc0, xr0, semA)
            scatter_chunk(xr0, pvc0)
            copy_idx(b0 + jnp.minimum(t0 + 2, nch - 1) * GCH, svc0, pvc0)
            issue_rows(svc0, xr0, semA)
            wait_rows(svc1, xr1, semB)
            scatter_chunk(xr1, pvc1)
            return carry

        lax.fori_loop(0, nch // 2, pairB, 0)
        wait_rows(svc0, xr0, semA)
        plsc.subcore_barrier()

        fbase = (c * R + r) * NODES + nbase
        for h in range(3):
            sz = GCH if h < 2 else NODES_PER_SUB - 2 * GCH
            for k in range((sz + LANES - 1) // LANES):
                sl = pl.ds(k * LANES, LANES)
                gph[sl] = g[pl.ds(h * GCH + k * LANES, LANES)]
            pltpu.sync_copy(acc_sh.at[gph.at[pl.ds(0, sz)]],
                            xr0.at[pl.ds(0, sz)])
            pltpu.sync_copy(xr0.at[pl.ds(0, sz)],
                            acc2_hbm.at[pl.ds(fbase + h * GCH, sz)])
        pltpu.sync_copy(cnt_sh.at[g], cv)
        pltpu.sync_copy(cv, cnt2_hbm.at[pl.ds(fbase, NODES_PER_SUB)])
        plsc.subcore_barrier()

    xb = wid * XN_PER_W
    pltpu.sync_copy(nodes_hbm.at[pl.ds(xb, XN_PER_W)], nv2)
    pltpu.async_copy(x_sp.at[nv2.at[pl.ds(0, GCH)]],
                     xr0, semA).wait()
    pltpu.sync_copy(xr0, xn_hbm.at[pl.ds(xb, GCH)])
    rem = XN_PER_W - GCH
    pltpu.async_copy(x_sp.at[nv2.at[pl.ds(GCH, rem)]],
                     xr0.at[pl.ds(0, rem)], semA).wait()
    pltpu.sync_copy(xr0.at[pl.ds(0, rem)],
                    xn_hbm.at[pl.ds(xb + GCH, rem)])


_SC_SCRATCH = [
    pltpu.VMEM_SHARED((SLOT_PAD,), jnp.int32),
    pltpu.VMEM_SHARED((X_ROWS, D), jnp.float32),
    pltpu.VMEM_SHARED((ACC_ROWS, D), jnp.float32),
    pltpu.VMEM_SHARED((ACC_ROWS,), jnp.float32),
    pltpu.VMEM_SHARED((NS * LIST_CAP,), jnp.int32),
    pltpu.VMEM((CHUNK,), jnp.int32),
    pltpu.VMEM((CHUNK,), jnp.int32),
    pltpu.VMEM((CHUNK,), jnp.int32),
    pltpu.VMEM((CHUNK,), jnp.int32),
    pltpu.VMEM((CHUNK,), jnp.int32),
    pltpu.VMEM((CHUNK,), jnp.int32),
    pltpu.VMEM((CHUNK,), jnp.int32),
    pltpu.VMEM((CHUNK,), jnp.int32),
    pltpu.VMEM((ALLOC_Q,), jnp.int32),
    pltpu.VMEM((ALLOC_Q,), jnp.int32),
    pltpu.VMEM((GCH,), jnp.int32),
    pltpu.VMEM((GCH,), jnp.int32),
    pltpu.VMEM((GCH,), jnp.int32),
    pltpu.VMEM((GCH,), jnp.int32),
    pltpu.VMEM((GCH,), jnp.int32),
    pltpu.VMEM((GCH, D), jnp.float32),
    pltpu.VMEM((GCH, D), jnp.float32),
    pltpu.VMEM((SLOT_PER_SUB,), jnp.float32),
    pltpu.VMEM((SLOT_PER_SUB,), jnp.int32),
    pltpu.VMEM((GCH,), jnp.float32),
    pltpu.VMEM((NODES_PER_SUB,), jnp.int32),
    pltpu.VMEM((NODES_PER_SUB,), jnp.int32),
    pltpu.VMEM((NODES_PER_SUB,), jnp.int32),
    pltpu.VMEM((GCH,), jnp.int32),
    pltpu.VMEM((NODES_PER_SUB,), jnp.float32),
    pltpu.VMEM((XN_PER_W,), jnp.int32),
    pltpu.SemaphoreType.DMA,
    pltpu.SemaphoreType.DMA,
    pltpu.SemaphoreType.DMA,
    pltpu.SemaphoreType.DMA,
]

_SC_OUT = (
    jax.ShapeDtypeStruct((NC * R * NODES, D), jnp.float32),
    jax.ShapeDtypeStruct((NC * R * NODES,), jnp.float32),
    jax.ShapeDtypeStruct((NODES, D), jnp.float32),
)

_sc_call_cached = None


def _sc_call(*args):
    global _sc_call_cached
    if _sc_call_cached is None:
        _sc_call_cached = pl.kernel(
            _sc_body,
            out_type=_SC_OUT,
            mesh=plsc.VectorSubcoreMesh(core_axis_name="c",
                                        subcore_axis_name="s",
                                        num_cores=NC, num_subcores=NS),
            scratch_types=_SC_SCRATCH,
        )
    return _sc_call_cached(*args)


def _tc_body(acc2_ref, cnt2_ref, xn_ref, wrel_ref, wroot_ref, brg_ref,
             wfc_ref, bfc_ref, out_ref):
    agg = jnp.zeros((NODES, D), jnp.float32)
    for r in range(R):
        acc_r = acc2_ref[0, r] + acc2_ref[1, r]
        cnt_r = cnt2_ref[0, r] + cnt2_ref[1, r]
        norm = 1.0 / jnp.maximum(cnt_r, 1.0)
        agg = agg + jnp.dot(acc_r * norm, wrel_ref[r],
                            preferred_element_type=jnp.float32)
    h = agg + jnp.dot(xn_ref[...], wroot_ref[...],
                      preferred_element_type=jnp.float32) + brg_ref[...]
    h = jnp.maximum(h, 0.0)
    comb = jnp.concatenate([h[:B], h[B:]], axis=1)
    out = jnp.dot(comb, wfc_ref[...],
                  preferred_element_type=jnp.float32) + bfc_ref[...]
    out_ref[...] = jnp.maximum(out, 0.0)


def _tc_call(acc2, cnt2, xn, W_rel, W_root, brg, W_fc, bfc):
    return pl.pallas_call(
        _tc_body,
        out_shape=jax.ShapeDtypeStruct((B, H), jnp.float32),
    )(acc2, cnt2, xn, W_rel, W_root, brg, W_fc, bfc)


def kernel(x, edge_index, edge_type, nest_tensor, food_tensor,
           W_rel, W_root, b_rgcn, W_fc, b_fc):
    src = edge_index[0].astype(jnp.int32)
    dst = edge_index[1].astype(jnp.int32)
    et = edge_type.astype(jnp.int32)
    nodes = jnp.concatenate([nest_tensor, food_tensor]).astype(jnp.int32)
    pad = E_PAD - E
    epk = src | (et << 16)
    epk = jnp.concatenate([epk, jnp.zeros((pad,), jnp.int32)])
    dst = jnp.concatenate([dst, jnp.full((pad,), N, jnp.int32)])
    x_pad = jnp.concatenate(
        [x, jnp.zeros((X_ROWS - N, D), jnp.float32)])

    acc2, cnt2, xn = _sc_call(x_pad, epk, dst, nodes)
    acc2 = acc2.reshape(NC, R, NODES, D)
    cnt2 = cnt2.reshape(NC, R, NODES, 1)
    return _tc_call(acc2, cnt2, xn, W_rel, W_root,
                    b_rgcn.reshape(1, D), W_fc, b_fc.reshape(1, H))


# speedup vs baseline: 2.9224x; 1.0003x over previous
import jax
import jax.numpy as jnp
from jax import lax
from jax.experimental import pallas as pl
from jax.experimental.pallas import tpu as pltpu
from jax.experimental.pallas import tpu_sc as plsc

N = 10000
E = 320000
D = 128
R = 4
H = 256
B = 1024

NC = 2
NS = 16
LANES = 16

NODES = 2 * B
SENTINEL = NODES
SLOT_PAD = 10240
X_ROWS = 10240
XSTAGE = X_ROWS // NS
ACC_ROWS = 2176
ROWS_PER_SUB = ACC_ROWS // NS
SLOT_PER_SUB = SLOT_PAD // NS
CHUNK = 128
CHUNKS_PER_SUB = 79
EDGES_PER_SUB = CHUNKS_PER_SUB * CHUNK
E_PAD = NC * NS * EDGES_PER_SUB
NODES_PER_SUB = NODES // NS
XN_PER_W = NODES // (NC * NS)
GCH = 48
ALLOC_Q = 2 * GCH
LIST = EDGES_PER_SUB + R * ALLOC_Q
TRASH = LIST
LIST_CAP = LIST + LANES
PAD_PACK = SENTINEL << 16


def _sc_body(x_hbm, epk_hbm, dst_hbm, nodes_hbm,
             acc2_hbm, cnt2_hbm, xn_hbm,
             slot_sh, x_sp, acc_sh, cnt_sh, cl_sh,
             dvA, dvB, epA, epB, sdA, sdB, pkbuf, posv,
             padbuf, padpos, pkC, svc0, svc1, pvc0, pvc1,
             xr0, xr1, zline, sbuf, ones, nv, vals, g, gph, cv, nv2,
             semA, semB, semDA, semDB):
    c = lax.axis_index("c")
    s = lax.axis_index("s")
    wid = s * NC + c
    list_base = s * LIST_CAP

    zero16 = jnp.zeros((LANES,), jnp.float32)
    iota16 = lax.iota(jnp.int32, LANES)
    sent16 = jnp.full((LANES,), SENTINEL, jnp.int32)
    one16 = jnp.ones((LANES,), jnp.float32)

    pltpu.async_copy(x_hbm.at[pl.ds(s * XSTAGE, XSTAGE)],
                     x_sp.at[pl.ds(s * XSTAGE, XSTAGE)], semA)

    for k in range(SLOT_PER_SUB // LANES):
        zline[pl.ds(k * LANES, LANES)] = zero16
        sbuf[pl.ds(k * LANES, LANES)] = sent16
    for k in range(GCH // LANES):
        ones[pl.ds(k * LANES, LANES)] = one16
    for k in range(ALLOC_Q // LANES):
        padbuf[pl.ds(k * LANES, LANES)] = jnp.full((LANES,), PAD_PACK,
                                                   jnp.int32)

    pltpu.sync_copy(sbuf, slot_sh.at[pl.ds(s * SLOT_PER_SUB, SLOT_PER_SUB)])
    pltpu.make_async_copy(x_hbm.at[pl.ds(0, XSTAGE)],
                          x_sp.at[pl.ds(0, XSTAGE)], semA).wait()

    plsc.subcore_barrier()

    nbase = s * NODES_PER_SUB
    pltpu.sync_copy(nodes_hbm.at[pl.ds(nbase, NODES_PER_SUB)], nv)
    for k in range(NODES_PER_SUB // LANES):
        vals[pl.ds(k * LANES, LANES)] = nbase + k * LANES + iota16
    pltpu.sync_copy(vals, slot_sh.at[nv])

    plsc.subcore_barrier()

    ebase = wid * EDGES_PER_SUB

    def load_ch(t, dvb, epb, sem):
        pltpu.async_copy(dst_hbm.at[pl.ds(ebase + t * CHUNK, CHUNK)],
                         dvb, sem)
        pltpu.async_copy(epk_hbm.at[pl.ds(ebase + t * CHUNK, CHUNK)],
                         epb, sem)

    def wait_ch(dvb, epb, sem):
        pltpu.make_async_copy(dst_hbm.at[pl.ds(0, CHUNK)], dvb, sem).wait()
        pltpu.make_async_copy(epk_hbm.at[pl.ds(0, CHUNK)], epb, sem).wait()

    def issue_sd(dvb, buf, sem):
        pltpu.async_copy(slot_sh.at[dvb], buf, sem)

    def wait_sd(buf, sem):
        pltpu.make_async_copy(slot_sh.at[dvA], buf, sem).wait()

    last = CHUNKS_PER_SUB - 1

    def prefix16(v):
        out = v
        for sh in (1, 2, 4, 8):
            idx = jnp.maximum(iota16 - sh, 0)
            shifted = jnp.take(out, idx, mode="wrap")
            out = out + jnp.where(iota16 >= sh, shifted, 0)
        return out

    def scan_pass(chunk_fn, carry0):
        load_ch(0, dvA, epA, semDA)
        wait_ch(dvA, epA, semDA)
        issue_sd(dvA, sdA, semA)
        load_ch(1, dvB, epB, semDB)

        def pair(i, carry):
            t0 = 2 * i
            wait_ch(dvB, epB, semDB)
            issue_sd(dvB, sdB, semB)
            wait_sd(sdA, semA)
            load_ch(jnp.minimum(t0 + 2, last), dvA, epA, semDA)
            carry = chunk_fn(sdA, epA, carry)
            wait_ch(dvA, epA, semDA)
            issue_sd(dvA, sdA, semA)
            wait_sd(sdB, semB)
            load_ch(jnp.minimum(t0 + 3, last), dvB, epB, semDB)
            carry = chunk_fn(sdB, epB, carry)
            return carry

        carry = lax.fori_loop(0, (CHUNKS_PER_SUB - 1) // 2, pair, carry0)
        wait_ch(dvB, epB, semDB)
        wait_sd(sdA, semA)
        return chunk_fn(sdA, epA, carry)

    def count_chunk(sdref, epref, carry):
        accA, accB = carry
        for k in range(CHUNK // LANES):
            sl = pl.ds(k * LANES, LANES)
            sd16 = sdref[sl]
            ev16 = lax.shift_right_logical(epref[sl], 16)
            valid = sd16 < SENTINEL
            fld = lax.shift_left(1, (ev16 & 1) * 16)
            accA = accA + jnp.where(valid & (ev16 < 2), fld, 0)
            accB = accB + jnp.where(valid & (ev16 >= 2), fld, 0)
        return (accA, accB)

    zi = jnp.zeros((LANES,), jnp.int32)
    sc_cnt = jax.named_scope("ph_count")
    sc_cnt.__enter__()
    accA, accB = scan_pass(count_chunk, (zi, zi))
    sc_cnt.__exit__(None, None, None)
    totA = prefix16(accA)[LANES - 1]
    totB = prefix16(accB)[LANES - 1]
    cnts = (totA & 0xFFFF, lax.shift_right_logical(totA, 16),
            totB & 0xFFFF, lax.shift_right_logical(totB, 16))
    allocs = []
    bases = []
    b = jnp.int32(0)
    for r in range(R):
        bases.append(b)
        a = ((jnp.maximum(cnts[r], 1) + ALLOC_Q - 1) // ALLOC_Q) * ALLOC_Q
        allocs.append(a)
        b = b + a

    def compact_chunk(sdref, epref, carry):
        cur0, cur1, cur2, cur3 = carry
        for k in range(CHUNK // LANES):
            sl = pl.ds(k * LANES, LANES)
            sd16 = sdref[sl]
            ep16 = epref[sl]
            ev16 = lax.shift_right_logical(ep16, 16)
            sv16 = ep16 & 0xFFFF
            valid = sd16 < SENTINEL
            fld = lax.shift_left(1, (ev16 & 1) * 16)
            indA = jnp.where(valid & (ev16 < 2), fld, 0)
            indB = jnp.where(valid & (ev16 >= 2), fld, 0)
            csA = prefix16(indA)
            csB = prefix16(indB)
            csel = jnp.where(ev16 < 2, csA, csB)
            cs = lax.shift_right_logical(csel, (ev16 & 1) * 16) & 0xFFFF
            base_l = jnp.where(ev16 < 2,
                               jnp.where(ev16 == 0, cur0, cur1),
                               jnp.where(ev16 == 2, cur2, cur3))
            pos16 = jnp.where(valid, list_base + base_l + cs - 1,
                              list_base + TRASH + iota16)
            pkbuf[sl] = sv16 | lax.shift_left(sd16, 16)
            posv[sl] = pos16
            tA = csA[LANES - 1]
            tB = csB[LANES - 1]
            cur0 = cur0 + (tA & 0xFFFF)
            cur1 = cur1 + lax.shift_right_logical(tA, 16)
            cur2 = cur2 + (tB & 0xFFFF)
            cur3 = cur3 + lax.shift_right_logical(tB, 16)
        pltpu.sync_copy(pkbuf, cl_sh.at[posv])
        return (cur0, cur1, cur2, cur3)

    sc_cp = jax.named_scope("ph_compact")
    sc_cp.__enter__()
    curs = scan_pass(compact_chunk,
                     (bases[0], bases[1], bases[2], bases[3]))
    sc_cp.__exit__(None, None, None)

    for r in range(R):
        lim = bases[r] + allocs[r]
        for k in range(ALLOC_Q // LANES):
            raw = curs[r] + k * LANES + iota16
            padpos[pl.ds(k * LANES, LANES)] = jnp.where(
                raw < lim, list_base + raw, list_base + TRASH + iota16)
        pltpu.sync_copy(padbuf, cl_sh.at[padpos])

    def copy_idx(off, svc, pvc):
        pltpu.sync_copy(cl_sh.at[pl.ds(list_base + off, GCH)], pkC)
        for k in range(GCH // LANES):
            sl = pl.ds(k * LANES, LANES)
            v = pkC[sl]
            svc[sl] = v & 0xFFFF
            pvc[sl] = lax.shift_right_logical(v, 16)

    def issue_rows(svc, buf, sem):
        pltpu.async_copy(x_sp.at[svc], buf, sem)

    def wait_rows(svc, buf, sem):
        pltpu.make_async_copy(x_sp.at[svc], buf, sem).wait()

    def scatter_chunk(xbuf, pvc):
        pltpu.sync_copy(xbuf, acc_sh.at[pvc], add=True)
        pltpu.sync_copy(ones, cnt_sh.at[pvc], add=True)

    def zero_xr0(i, carry):
        for k in range(D // LANES):
            xr0[i, pl.ds(k * LANES, LANES)] = zero16
        return carry

    base_r = s * ROWS_PER_SUB
    pltpu.sync_copy(slot_sh.at[nv], g)

    sc_ps = jax.named_scope("ph_passes")
    sc_ps.__enter__()
    for r in range(R):
        lax.fori_loop(0, GCH, zero_xr0, 0)
        pltpu.sync_copy(xr0, acc_sh.at[pl.ds(base_r, GCH)])
        pltpu.sync_copy(xr0, acc_sh.at[pl.ds(base_r + GCH, GCH)])
        pltpu.sync_copy(xr0.at[pl.ds(0, ROWS_PER_SUB - 2 * GCH)],
                        acc_sh.at[pl.ds(base_r + 2 * GCH,
                                        ROWS_PER_SUB - 2 * GCH)])
        pltpu.sync_copy(zline.at[pl.ds(0, ROWS_PER_SUB)],
                        cnt_sh.at[pl.ds(base_r, ROWS_PER_SUB)])
        plsc.subcore_barrier()

        nch = allocs[r] // GCH
        b0 = bases[r]
        copy_idx(b0, svc0, pvc0)
        issue_rows(svc0, xr0, semA)

        def pairB(i, carry):
            t0 = 2 * i
            copy_idx(b0 + (t0 + 1) * GCH, svc1, pvc1)
            issue_rows(svc1, xr1, semB)
            wait_rows(svc0, xr0, semA)
            scatter_chunk(xr0, pvc0)
            copy_idx(b0 + jnp.minimum(t0 + 2, nch - 1) * GCH, svc0, pvc0)
            issue_rows(svc0, xr0, semA)
            wait_rows(svc1, xr1, semB)
            scatter_chunk(xr1, pvc1)
            return carry

        lax.fori_loop(0, nch // 2, pairB, 0)
        wait_rows(svc0, xr0, semA)
        plsc.subcore_barrier()

        fbase = (c * R + r) * NODES + nbase
        for h in range(3):
            sz = GCH if h < 2 else NODES_PER_SUB - 2 * GCH
            for k in range((sz + LANES - 1) // LANES):
                sl = pl.ds(k * LANES, LANES)
                gph[sl] = g[pl.ds(h * GCH + k * LANES, LANES)]
            pltpu.sync_copy(acc_sh.at[gph.at[pl.ds(0, sz)]],
                            xr0.at[pl.ds(0, sz)])
            pltpu.sync_copy(xr0.at[pl.ds(0, sz)],
                            acc2_hbm.at[pl.ds(fbase + h * GCH, sz)])
        pltpu.sync_copy(cnt_sh.at[g], cv)
        pltpu.sync_copy(cv, cnt2_hbm.at[pl.ds(fbase, NODES_PER_SUB)])
        plsc.subcore_barrier()
    sc_ps.__exit__(None, None, None)

    xb = wid * XN_PER_W
    pltpu.sync_copy(nodes_hbm.at[pl.ds(xb, XN_PER_W)], nv2)
    pltpu.async_copy(x_sp.at[nv2.at[pl.ds(0, GCH)]],
                     xr0, semA).wait()
    pltpu.sync_copy(xr0, xn_hbm.at[pl.ds(xb, GCH)])
    rem = XN_PER_W - GCH
    pltpu.async_copy(x_sp.at[nv2.at[pl.ds(GCH, rem)]],
                     xr0.at[pl.ds(0, rem)], semA).wait()
    pltpu.sync_copy(xr0.at[pl.ds(0, rem)],
                    xn_hbm.at[pl.ds(xb + GCH, rem)])


_SC_SCRATCH = [
    pltpu.VMEM_SHARED((SLOT_PAD,), jnp.int32),
    pltpu.VMEM_SHARED((X_ROWS, D), jnp.float32),
    pltpu.VMEM_SHARED((ACC_ROWS, D), jnp.float32),
    pltpu.VMEM_SHARED((ACC_ROWS,), jnp.float32),
    pltpu.VMEM_SHARED((NS * LIST_CAP,), jnp.int32),
    pltpu.VMEM((CHUNK,), jnp.int32),
    pltpu.VMEM((CHUNK,), jnp.int32),
    pltpu.VMEM((CHUNK,), jnp.int32),
    pltpu.VMEM((CHUNK,), jnp.int32),
    pltpu.VMEM((CHUNK,), jnp.int32),
    pltpu.VMEM((CHUNK,), jnp.int32),
    pltpu.VMEM((CHUNK,), jnp.int32),
    pltpu.VMEM((CHUNK,), jnp.int32),
    pltpu.VMEM((ALLOC_Q,), jnp.int32),
    pltpu.VMEM((ALLOC_Q,), jnp.int32),
    pltpu.VMEM((GCH,), jnp.int32),
    pltpu.VMEM((GCH,), jnp.int32),
    pltpu.VMEM((GCH,), jnp.int32),
    pltpu.VMEM((GCH,), jnp.int32),
    pltpu.VMEM((GCH,), jnp.int32),
    pltpu.VMEM((GCH, D), jnp.float32),
    pltpu.VMEM((GCH, D), jnp.float32),
    pltpu.VMEM((SLOT_PER_SUB,), jnp.float32),
    pltpu.VMEM((SLOT_PER_SUB,), jnp.int32),
    pltpu.VMEM((GCH,), jnp.float32),
    pltpu.VMEM((NODES_PER_SUB,), jnp.int32),
    pltpu.VMEM((NODES_PER_SUB,), jnp.int32),
    pltpu.VMEM((NODES_PER_SUB,), jnp.int32),
    pltpu.VMEM((GCH,), jnp.int32),
    pltpu.VMEM((NODES_PER_SUB,), jnp.float32),
    pltpu.VMEM((XN_PER_W,), jnp.int32),
    pltpu.SemaphoreType.DMA,
    pltpu.SemaphoreType.DMA,
    pltpu.SemaphoreType.DMA,
    pltpu.SemaphoreType.DMA,
]

_SC_OUT = (
    jax.ShapeDtypeStruct((NC * R * NODES, D), jnp.float32),
    jax.ShapeDtypeStruct((NC * R * NODES,), jnp.float32),
    jax.ShapeDtypeStruct((NODES, D), jnp.float32),
)

_sc_call_cached = None


def _sc_call(*args):
    global _sc_call_cached
    if _sc_call_cached is None:
        _sc_call_cached = pl.kernel(
            _sc_body,
            out_type=_SC_OUT,
            mesh=plsc.VectorSubcoreMesh(core_axis_name="c",
                                        subcore_axis_name="s",
                                        num_cores=NC, num_subcores=NS),
            scratch_types=_SC_SCRATCH,
        )
    return _sc_call_cached(*args)


def _tc_body(acc2_ref, cnt2_ref, xn_ref, wrel_ref, wroot_ref, brg_ref,
             wfc_ref, bfc_ref, out_ref):
    agg = jnp.zeros((NODES, D), jnp.float32)
    for r in range(R):
        acc_r = acc2_ref[0, r] + acc2_ref[1, r]
        cnt_r = cnt2_ref[0, r] + cnt2_ref[1, r]
        norm = 1.0 / jnp.maximum(cnt_r, 1.0)
        agg = agg + jnp.dot(acc_r * norm, wrel_ref[r],
                            preferred_element_type=jnp.float32)
    h = agg + jnp.dot(xn_ref[...], wroot_ref[...],
                      preferred_element_type=jnp.float32) + brg_ref[...]
    h = jnp.maximum(h, 0.0)
    comb = jnp.concatenate([h[:B], h[B:]], axis=1)
    out = jnp.dot(comb, wfc_ref[...],
                  preferred_element_type=jnp.float32) + bfc_ref[...]
    out_ref[...] = jnp.maximum(out, 0.0)


def _tc_call(acc2, cnt2, xn, W_rel, W_root, brg, W_fc, bfc):
    return pl.pallas_call(
        _tc_body,
        out_shape=jax.ShapeDtypeStruct((B, H), jnp.float32),
    )(acc2, cnt2, xn, W_rel, W_root, brg, W_fc, bfc)


def kernel(x, edge_index, edge_type, nest_tensor, food_tensor,
           W_rel, W_root, b_rgcn, W_fc, b_fc):
    src = edge_index[0].astype(jnp.int32)
    dst = edge_index[1].astype(jnp.int32)
    et = edge_type.astype(jnp.int32)
    nodes = jnp.concatenate([nest_tensor, food_tensor]).astype(jnp.int32)
    pad = E_PAD - E
    epk = src | (et << 16)
    epk = jnp.concatenate([epk, jnp.zeros((pad,), jnp.int32)])
    dst = jnp.concatenate([dst, jnp.full((pad,), N, jnp.int32)])
    x_pad = jnp.concatenate(
        [x, jnp.zeros((X_ROWS - N, D), jnp.float32)])

    acc2, cnt2, xn = _sc_call(x_pad, epk, dst, nodes)
    acc2 = acc2.reshape(NC, R, NODES, D)
    cnt2 = cnt2.reshape(NC, R, NODES, 1)
    return _tc_call(acc2, cnt2, xn, W_rel, W_root,
                    b_rgcn.reshape(1, D), W_fc, b_fc.reshape(1, H))


# speedup vs baseline: 3.2829x; 1.1234x over previous
import jax
import jax.numpy as jnp
from jax import lax
from jax.experimental import pallas as pl
from jax.experimental.pallas import tpu as pltpu
from jax.experimental.pallas import tpu_sc as plsc

N = 10000
E = 320000
D = 128
R = 4
H = 256
B = 1024

NC = 2
NS = 16
LANES = 16

NODES = 2 * B
SENTINEL = NODES
SLOT_PAD = 10240
X_ROWS = 10240
XSTAGE = X_ROWS // NS
ACC_ROWS = 2176
ROWS_PER_SUB = ACC_ROWS // NS
SLOT_PER_SUB = SLOT_PAD // NS
CHUNK = 128
CHUNKS_PER_SUB = 79
EDGES_PER_SUB = CHUNKS_PER_SUB * CHUNK
E_PAD = NC * NS * EDGES_PER_SUB
NODES_PER_SUB = NODES // NS
XN_PER_W = NODES // (NC * NS)
GCH = 48
ALLOC_Q = 2 * GCH
LIST = EDGES_PER_SUB + 2 * R * ALLOC_Q + ALLOC_Q
TRASH = LIST
LIST_CAP = LIST + LANES
PAD_PACK = SENTINEL << 16


def _sc_body(x_hbm, epk_hbm, dst_hbm, nodes_hbm,
             acc2_hbm, cnt2_hbm, xn_hbm,
             slot_sh, x_sp, acc_sh, cnt_sh, cl_sh,
             dvA, dvB, epA, epB, sdA, sdB, pkbuf, posv,
             padbuf, padpos, pkC, svc0, svc1, pvc0, pvc1,
             xr0, xr1, zline, sbuf, ones, nv, vals, g, gph, cv, nv2,
             semA, semB, semDA, semDB):
    c = lax.axis_index("c")
    s = lax.axis_index("s")
    wid = s * NC + c
    list_base = s * LIST_CAP

    zero16 = jnp.zeros((LANES,), jnp.float32)
    iota16 = lax.iota(jnp.int32, LANES)
    sent16 = jnp.full((LANES,), SENTINEL, jnp.int32)
    one16 = jnp.ones((LANES,), jnp.float32)

    pltpu.async_copy(x_hbm.at[pl.ds(s * XSTAGE, XSTAGE)],
                     x_sp.at[pl.ds(s * XSTAGE, XSTAGE)], semA)

    for k in range(SLOT_PER_SUB // LANES):
        zline[pl.ds(k * LANES, LANES)] = zero16
        sbuf[pl.ds(k * LANES, LANES)] = sent16
    for k in range(GCH // LANES):
        ones[pl.ds(k * LANES, LANES)] = one16
    for k in range(ALLOC_Q // LANES):
        padbuf[pl.ds(k * LANES, LANES)] = jnp.full((LANES,), PAD_PACK,
                                                   jnp.int32)

    pltpu.sync_copy(sbuf, slot_sh.at[pl.ds(s * SLOT_PER_SUB, SLOT_PER_SUB)])
    pltpu.make_async_copy(x_hbm.at[pl.ds(0, XSTAGE)],
                          x_sp.at[pl.ds(0, XSTAGE)], semA).wait()

    plsc.subcore_barrier()

    nbase = s * NODES_PER_SUB
    pltpu.sync_copy(nodes_hbm.at[pl.ds(nbase, NODES_PER_SUB)], nv)
    for k in range(NODES_PER_SUB // LANES):
        vals[pl.ds(k * LANES, LANES)] = nbase + k * LANES + iota16
    pltpu.sync_copy(vals, slot_sh.at[nv])

    plsc.subcore_barrier()

    ebase = wid * EDGES_PER_SUB

    def load_ch(t, dvb, epb, sem):
        pltpu.async_copy(dst_hbm.at[pl.ds(ebase + t * CHUNK, CHUNK)],
                         dvb, sem)
        pltpu.async_copy(epk_hbm.at[pl.ds(ebase + t * CHUNK, CHUNK)],
                         epb, sem)

    def wait_ch(dvb, epb, sem):
        pltpu.make_async_copy(dst_hbm.at[pl.ds(0, CHUNK)], dvb, sem).wait()
        pltpu.make_async_copy(epk_hbm.at[pl.ds(0, CHUNK)], epb, sem).wait()

    def issue_sd(dvb, buf, sem):
        pltpu.async_copy(slot_sh.at[dvb], buf, sem)

    def wait_sd(buf, sem):
        pltpu.make_async_copy(slot_sh.at[dvA], buf, sem).wait()

    last = CHUNKS_PER_SUB - 1

    def prefix16(v):
        out = v
        for sh in (1, 2, 4, 8):
            idx = jnp.maximum(iota16 - sh, 0)
            shifted = jnp.take(out, idx, mode="wrap")
            out = out + jnp.where(iota16 >= sh, shifted, 0)
        return out

    def scan_pass(chunk_fn, carry0):
        load_ch(0, dvA, epA, semDA)
        wait_ch(dvA, epA, semDA)
        issue_sd(dvA, sdA, semA)
        load_ch(1, dvB, epB, semDB)

        def pair(i, carry):
            t0 = 2 * i
            wait_ch(dvB, epB, semDB)
            issue_sd(dvB, sdB, semB)
            wait_sd(sdA, semA)
            load_ch(jnp.minimum(t0 + 2, last), dvA, epA, semDA)
            carry = chunk_fn(sdA, epA, carry)
            wait_ch(dvA, epA, semDA)
            issue_sd(dvA, sdA, semA)
            wait_sd(sdB, semB)
            load_ch(jnp.minimum(t0 + 3, last), dvB, epB, semDB)
            carry = chunk_fn(sdB, epB, carry)
            return carry

        carry = lax.fori_loop(0, (CHUNKS_PER_SUB - 1) // 2, pair, carry0)
        wait_ch(dvB, epB, semDB)
        wait_sd(sdA, semA)
        return chunk_fn(sdA, epA, carry)

    def count_chunk(epref, carry):
        accA, accB = carry
        for k in range(CHUNK // LANES):
            sl = pl.ds(k * LANES, LANES)
            ev16 = lax.shift_right_logical(epref[sl], 16)
            fld = lax.shift_left(1, (ev16 & 1) * 16)
            accA = accA + jnp.where(ev16 < 2, fld, 0)
            accB = accB + jnp.where(ev16 >= 2, fld, 0)
        return (accA, accB)

    zi = jnp.zeros((LANES,), jnp.int32)
    sc_cnt = jax.named_scope("ph_count")
    sc_cnt.__enter__()

    def load_ep(t, epb, sem):
        pltpu.async_copy(epk_hbm.at[pl.ds(ebase + t * CHUNK, CHUNK)],
                         epb, sem)

    def wait_ep(epb, sem):
        pltpu.make_async_copy(epk_hbm.at[pl.ds(0, CHUNK)], epb, sem).wait()

    load_ep(0, epA, semDA)

    def cpair(i, carry):
        t0 = 2 * i
        load_ep(t0 + 1, epB, semDB)
        wait_ep(epA, semDA)
        carry = count_chunk(epA, carry)
        load_ep(jnp.minimum(t0 + 2, last), epA, semDA)
        wait_ep(epB, semDB)
        carry = count_chunk(epB, carry)
        return carry

    accA, accB = lax.fori_loop(0, (CHUNKS_PER_SUB - 1) // 2, cpair,
                               (zi, zi))
    wait_ep(epA, semDA)
    accA, accB = count_chunk(epA, (accA, accB))
    sc_cnt.__exit__(None, None, None)

    totA = prefix16(accA)[LANES - 1]
    totB = prefix16(accB)[LANES - 1]
    cnts = (totA & 0xFFFF, lax.shift_right_logical(totA, 16),
            totB & 0xFFFF, lax.shift_right_logical(totB, 16))
    allocs = []
    bases = []
    b = jnp.int32(0)
    for r in range(R):
        bases.append(b)
        a = (((jnp.maximum(cnts[r], 1) + ALLOC_Q - 1) // ALLOC_Q)
             * ALLOC_Q + ALLOC_Q)
        allocs.append(a)
        b = b + a

    def compact_chunk(sdref, epref, carry):
        cur0, cur1, cur2, cur3 = carry
        for k in range(CHUNK // LANES):
            sl = pl.ds(k * LANES, LANES)
            sd16 = sdref[sl]
            ep16 = epref[sl]
            ev16 = lax.shift_right_logical(ep16, 16)
            sv16 = ep16 & 0xFFFF
            valid = sd16 < SENTINEL
            fld = lax.shift_left(1, (ev16 & 1) * 16)
            indA = jnp.where(valid & (ev16 < 2), fld, 0)
            indB = jnp.where(valid & (ev16 >= 2), fld, 0)
            csA = prefix16(indA)
            csB = prefix16(indB)
            csel = jnp.where(ev16 < 2, csA, csB)
            cs = lax.shift_right_logical(csel, (ev16 & 1) * 16) & 0xFFFF
            base_l = jnp.where(ev16 < 2,
                               jnp.where(ev16 == 0, cur0, cur1),
                               jnp.where(ev16 == 2, cur2, cur3))
            pos16 = jnp.where(valid, list_base + base_l + cs - 1,
                              list_base + TRASH + iota16)
            pkbuf[sl] = sv16 | lax.shift_left(sd16, 16)
            posv[sl] = pos16
            tA = csA[LANES - 1]
            tB = csB[LANES - 1]
            cur0 = cur0 + (tA & 0xFFFF)
            cur1 = cur1 + lax.shift_right_logical(tA, 16)
            cur2 = cur2 + (tB & 0xFFFF)
            cur3 = cur3 + lax.shift_right_logical(tB, 16)
        pltpu.sync_copy(pkbuf, cl_sh.at[posv])
        return (cur0, cur1, cur2, cur3)

    sc_cp = jax.named_scope("ph_compact")
    sc_cp.__enter__()
    curs = scan_pass(compact_chunk,
                     (bases[0], bases[1], bases[2], bases[3]))
    sc_cp.__exit__(None, None, None)

    for r in range(R):
        lim = bases[r] + allocs[r]
        for k in range(ALLOC_Q // LANES):
            raw = curs[r] + k * LANES + iota16
            padpos[pl.ds(k * LANES, LANES)] = jnp.where(
                raw < lim, list_base + raw, list_base + TRASH + iota16)
        pltpu.sync_copy(padbuf, cl_sh.at[padpos])

    def copy_idx(off, svc, pvc):
        pltpu.sync_copy(cl_sh.at[pl.ds(list_base + off, GCH)], pkC)
        for k in range(GCH // LANES):
            sl = pl.ds(k * LANES, LANES)
            v = pkC[sl]
            svc[sl] = v & 0xFFFF
            pvc[sl] = lax.shift_right_logical(v, 16)

    def issue_rows(svc, buf, sem):
        pltpu.async_copy(x_sp.at[svc], buf, sem)

    def wait_rows(svc, buf, sem):
        pltpu.make_async_copy(x_sp.at[svc], buf, sem).wait()

    def scatter_chunk(xbuf, pvc):
        pltpu.sync_copy(xbuf, acc_sh.at[pvc], add=True)
        pltpu.sync_copy(ones, cnt_sh.at[pvc], add=True)

    def zero_xr0(i, carry):
        for k in range(D // LANES):
            xr0[i, pl.ds(k * LANES, LANES)] = zero16
        return carry

    base_r = s * ROWS_PER_SUB
    pltpu.sync_copy(slot_sh.at[nv], g)

    sc_ps = jax.named_scope("ph_passes")
    sc_ps.__enter__()
    for r in range(R):
        lax.fori_loop(0, GCH, zero_xr0, 0)
        pltpu.sync_copy(xr0, acc_sh.at[pl.ds(base_r, GCH)])
        pltpu.sync_copy(xr0, acc_sh.at[pl.ds(base_r + GCH, GCH)])
        pltpu.sync_copy(xr0.at[pl.ds(0, ROWS_PER_SUB - 2 * GCH)],
                        acc_sh.at[pl.ds(base_r + 2 * GCH,
                                        ROWS_PER_SUB - 2 * GCH)])
        pltpu.sync_copy(zline.at[pl.ds(0, ROWS_PER_SUB)],
                        cnt_sh.at[pl.ds(base_r, ROWS_PER_SUB)])
        plsc.subcore_barrier()

        valid_r = curs[r] - bases[r]
        nc1 = (valid_r + GCH - 1) // GCH
        nch = jnp.maximum(nc1 + (nc1 % 2), 2)
        b0 = bases[r]
        copy_idx(b0, svc0, pvc0)
        issue_rows(svc0, xr0, semA)

        def pairB(i, carry):
            t0 = 2 * i
            copy_idx(b0 + (t0 + 1) * GCH, svc1, pvc1)
            issue_rows(svc1, xr1, semB)
            wait_rows(svc0, xr0, semA)
            scatter_chunk(xr0, pvc0)
            copy_idx(b0 + jnp.minimum(t0 + 2, nch - 1) * GCH, svc0, pvc0)
            issue_rows(svc0, xr0, semA)
            wait_rows(svc1, xr1, semB)
            scatter_chunk(xr1, pvc1)
            return carry

        lax.fori_loop(0, nch // 2, pairB, 0)
        wait_rows(svc0, xr0, semA)
        plsc.subcore_barrier()

        fbase = (c * R + r) * NODES + nbase
        for h in range(3):
            sz = GCH if h < 2 else NODES_PER_SUB - 2 * GCH
            for k in range((sz + LANES - 1) // LANES):
                sl = pl.ds(k * LANES, LANES)
                gph[sl] = g[pl.ds(h * GCH + k * LANES, LANES)]
            pltpu.sync_copy(acc_sh.at[gph.at[pl.ds(0, sz)]],
                            xr0.at[pl.ds(0, sz)])
            pltpu.sync_copy(xr0.at[pl.ds(0, sz)],
                            acc2_hbm.at[pl.ds(fbase + h * GCH, sz)])
        pltpu.sync_copy(cnt_sh.at[g], cv)
        pltpu.sync_copy(cv, cnt2_hbm.at[pl.ds(fbase, NODES_PER_SUB)])
        plsc.subcore_barrier()
    sc_ps.__exit__(None, None, None)

    xb = wid * XN_PER_W
    pltpu.sync_copy(nodes_hbm.at[pl.ds(xb, XN_PER_W)], nv2)
    pltpu.async_copy(x_sp.at[nv2.at[pl.ds(0, GCH)]],
                     xr0, semA).wait()
    pltpu.sync_copy(xr0, xn_hbm.at[pl.ds(xb, GCH)])
    rem = XN_PER_W - GCH
    pltpu.async_copy(x_sp.at[nv2.at[pl.ds(GCH, rem)]],
                     xr0.at[pl.ds(0, rem)], semA).wait()
    pltpu.sync_copy(xr0.at[pl.ds(0, rem)],
                    xn_hbm.at[pl.ds(xb + GCH, rem)])


_SC_SCRATCH = [
    pltpu.VMEM_SHARED((SLOT_PAD,), jnp.int32),
    pltpu.VMEM_SHARED((X_ROWS, D), jnp.float32),
    pltpu.VMEM_SHARED((ACC_ROWS, D), jnp.float32),
    pltpu.VMEM_SHARED((ACC_ROWS,), jnp.float32),
    pltpu.VMEM_SHARED((NS * LIST_CAP,), jnp.int32),
    pltpu.VMEM((CHUNK,), jnp.int32),
    pltpu.VMEM((CHUNK,), jnp.int32),
    pltpu.VMEM((CHUNK,), jnp.int32),
    pltpu.VMEM((CHUNK,), jnp.int32),
    pltpu.VMEM((CHUNK,), jnp.int32),
    pltpu.VMEM((CHUNK,), jnp.int32),
    pltpu.VMEM((CHUNK,), jnp.int32),
    pltpu.VMEM((CHUNK,), jnp.int32),
    pltpu.VMEM((ALLOC_Q,), jnp.int32),
    pltpu.VMEM((ALLOC_Q,), jnp.int32),
    pltpu.VMEM((GCH,), jnp.int32),
    pltpu.VMEM((GCH,), jnp.int32),
    pltpu.VMEM((GCH,), jnp.int32),
    pltpu.VMEM((GCH,), jnp.int32),
    pltpu.VMEM((GCH,), jnp.int32),
    pltpu.VMEM((GCH, D), jnp.float32),
    pltpu.VMEM((GCH, D), jnp.float32),
    pltpu.VMEM((SLOT_PER_SUB,), jnp.float32),
    pltpu.VMEM((SLOT_PER_SUB,), jnp.int32),
    pltpu.VMEM((GCH,), jnp.float32),
    pltpu.VMEM((NODES_PER_SUB,), jnp.int32),
    pltpu.VMEM((NODES_PER_SUB,), jnp.int32),
    pltpu.VMEM((NODES_PER_SUB,), jnp.int32),
    pltpu.VMEM((GCH,), jnp.int32),
    pltpu.VMEM((NODES_PER_SUB,), jnp.float32),
    pltpu.VMEM((XN_PER_W,), jnp.int32),
    pltpu.SemaphoreType.DMA,
    pltpu.SemaphoreType.DMA,
    pltpu.SemaphoreType.DMA,
    pltpu.SemaphoreType.DMA,
]

_SC_OUT = (
    jax.ShapeDtypeStruct((NC * R * NODES, D), jnp.float32),
    jax.ShapeDtypeStruct((NC * R * NODES,), jnp.float32),
    jax.ShapeDtypeStruct((NODES, D), jnp.float32),
)

_sc_call_cached = None


def _sc_call(*args):
    global _sc_call_cached
    if _sc_call_cached is None:
        _sc_call_cached = pl.kernel(
            _sc_body,
            out_type=_SC_OUT,
            mesh=plsc.VectorSubcoreMesh(core_axis_name="c",
                                        subcore_axis_name="s",
                                        num_cores=NC, num_subcores=NS),
            scratch_types=_SC_SCRATCH,
        )
    return _sc_call_cached(*args)


def _tc_body(acc2_ref, cnt2_ref, xn_ref, wrel_ref, wroot_ref, brg_ref,
             wfc_ref, bfc_ref, out_ref):
    agg = jnp.zeros((NODES, D), jnp.float32)
    for r in range(R):
        acc_r = acc2_ref[0, r] + acc2_ref[1, r]
        cnt_r = cnt2_ref[0, r] + cnt2_ref[1, r]
        norm = 1.0 / jnp.maximum(cnt_r, 1.0)
        agg = agg + jnp.dot(acc_r * norm, wrel_ref[r],
                            preferred_element_type=jnp.float32)
    h = agg + jnp.dot(xn_ref[...], wroot_ref[...],
                      preferred_element_type=jnp.float32) + brg_ref[...]
    h = jnp.maximum(h, 0.0)
    comb = jnp.concatenate([h[:B], h[B:]], axis=1)
    out = jnp.dot(comb, wfc_ref[...],
                  preferred_element_type=jnp.float32) + bfc_ref[...]
    out_ref[...] = jnp.maximum(out, 0.0)


def _tc_call(acc2, cnt2, xn, W_rel, W_root, brg, W_fc, bfc):
    return pl.pallas_call(
        _tc_body,
        out_shape=jax.ShapeDtypeStruct((B, H), jnp.float32),
    )(acc2, cnt2, xn, W_rel, W_root, brg, W_fc, bfc)


def kernel(x, edge_index, edge_type, nest_tensor, food_tensor,
           W_rel, W_root, b_rgcn, W_fc, b_fc):
    src = edge_index[0].astype(jnp.int32)
    dst = edge_index[1].astype(jnp.int32)
    et = edge_type.astype(jnp.int32)
    nodes = jnp.concatenate([nest_tensor, food_tensor]).astype(jnp.int32)
    pad = E_PAD - E
    epk = src | (et << 16)
    epk = jnp.concatenate([epk, jnp.zeros((pad,), jnp.int32)])
    dst = jnp.concatenate([dst, jnp.full((pad,), N, jnp.int32)])
    x_pad = jnp.concatenate(
        [x, jnp.zeros((X_ROWS - N, D), jnp.float32)])

    acc2, cnt2, xn = _sc_call(x_pad, epk, dst, nodes)
    acc2 = acc2.reshape(NC, R, NODES, D)
    cnt2 = cnt2.reshape(NC, R, NODES, 1)
    return _tc_call(acc2, cnt2, xn, W_rel, W_root,
                    b_rgcn.reshape(1, D), W_fc, b_fc.reshape(1, H))
